# baseline (speedup 1.0000x reference)
import functools

import jax
import jax.numpy as jnp
from jax import lax
from jax.experimental import pallas as pl
from jax.experimental.pallas import tpu as pltpu
from jax.experimental.pallas import tpu_sc as plsc

NUM_ITEMS = 1000000
CF_DIM = 64

NC = 2
NS = 16
NW = NC * NS

NB = 16384
NSLOT = 50
BLK = 128
N_BLOCKS = NSLOT * (NB // BLK)
BLK_PER_W = N_BLOCKS // NW
BH = NB // BLK
PITCH = BLK + 1


def _sc_gather(idx2, table):
  mesh = plsc.VectorSubcoreMesh(core_axis_name="c", subcore_axis_name="s")

  @functools.partial(
      pl.kernel,
      out_type=jax.ShapeDtypeStruct((NSLOT, 8, BH, 8, BLK), jnp.float32),
      mesh=mesh,
      compiler_params=pltpu.CompilerParams(
          use_tc_tiling_on_sc=False, needs_layout_passes=False),
      scratch_types=[
          pltpu.VMEM((BLK_PER_W, BLK), jnp.int32),
          *[pltpu.VMEM((BLK, CF_DIM), jnp.float32) for _ in range(3)],
          *[pltpu.VMEM((CF_DIM, PITCH), jnp.float32) for _ in range(3)],
          *[pltpu.SemaphoreType.DMA for _ in range(6)],
      ],
  )
  def k(idx_hbm, table_hbm, out_hbm, idx_v, rows0, rows1, rows2,
        t0, t1, t2, g0, g1, g2, o0, o1, o2):
    rows = (rows0, rows1, rows2)
    tiles = (t0, t1, t2)
    gsem = (g0, g1, g2)
    osem = (o0, o1, o2)
    wid = lax.axis_index("s") * NC + lax.axis_index("c")
    base_blk = wid * BLK_PER_W

    pltpu.sync_copy(idx_hbm.at[pl.ds(base_blk, BLK_PER_W)], idx_v)

    iot = lax.iota(jnp.int32, 16)
    rvecs = [iot + 16 * dg for dg in range(4)]

    def transpose_block(src, dst):
      for b0 in range(0, BLK, 4):
        vs = [
            src[b0 + i, pl.ds(16 * dg, 16)]
            for i in range(4)
            for dg in range(4)
        ]
        for i in range(4):
          bvec = jnp.full((16,), b0 + i, jnp.int32)
          for dg in range(4):
            plsc.store_scatter(dst, [rvecs[dg], bvec], vs[i * 4 + dg])

    def out_copies(src, s, bh, sem):
      for dh in range(8):
        pltpu.async_copy(
            src.at[pl.ds(8 * dh, 8), pl.ds(0, BLK)],
            out_hbm.at[s, dh, bh], sem)

    def drain_out(src, sem):
      for dh in range(8):
        pltpu.make_async_copy(
            src.at[pl.ds(8 * dh, 8), pl.ds(0, BLK)],
            out_hbm.at[0, 0, 0], sem).wait()

    def when_(cond, fn):
      if isinstance(cond, bool):
        if cond:
          fn()
      else:
        pl.when(cond)(fn)

    def step(j, p):
      def fire_ahead():
        pltpu.async_copy(
            table_hbm.at[idx_v.at[j + 2]], rows[(p + 2) % 3],
            gsem[(p + 2) % 3])
        return None

      when_(j + 2 < BLK_PER_W, fire_ahead)

      pltpu.make_async_copy(
          table_hbm.at[pl.ds(0, BLK)], rows[p], gsem[p]).wait()

      when_(j >= 3, lambda: drain_out(tiles[p], osem[p]))

      transpose_block(rows[p], tiles[p])

      blk = base_blk + j
      s = blk // BH
      bh = lax.rem(blk, BH)
      out_copies(tiles[p], s, bh, osem[p])

    pltpu.async_copy(table_hbm.at[idx_v.at[0]], rows[0], gsem[0])
    pltpu.async_copy(table_hbm.at[idx_v.at[1]], rows[1], gsem[1])

    @pl.loop(0, (BLK_PER_W - 2) // 3)
    def _(g):
      for p in range(3):
        step(3 * g + p, p)

    step(BLK_PER_W - 2, 0)
    step(BLK_PER_W - 1, 1)

    for p in range(3):
      drain_out(tiles[p], osem[p])

  return k(idx2, table)


@jax.jit
def kernel(item_indices, item_embeddings):
  idx2 = item_indices.astype(jnp.int32).T.reshape(N_BLOCKS, BLK)
  out5 = _sc_gather(idx2, item_embeddings)
  return out5.transpose((2, 4, 0, 1, 3)).reshape(NB, NSLOT, CF_DIM)


# speedup vs baseline: 1.0865x; 1.0865x over previous
import functools

import jax
import jax.numpy as jnp
from jax import lax
from jax.experimental import pallas as pl
from jax.experimental.pallas import tpu as pltpu
from jax.experimental.pallas import tpu_sc as plsc

NUM_ITEMS = 1000000
CF_DIM = 64

NC = 2
NS = 16
NW = NC * NS

NB = 16384
NSLOT = 50
BLK = 128
N_BLOCKS = NSLOT * (NB // BLK)
BLK_PER_W = N_BLOCKS // NW
BH = NB // BLK
PITCH = BLK + 1


def _sc_gather(idx2, table):
  mesh = plsc.VectorSubcoreMesh(core_axis_name="c", subcore_axis_name="s")

  @functools.partial(
      pl.kernel,
      out_type=jax.ShapeDtypeStruct((NSLOT, 8, BH, 8, BLK), jnp.float32),
      mesh=mesh,
      compiler_params=pltpu.CompilerParams(
          use_tc_tiling_on_sc=False, needs_layout_passes=False),
      scratch_types=[
          pltpu.VMEM((BLK_PER_W, BLK), jnp.int32),
          *[pltpu.VMEM((BLK, CF_DIM), jnp.float32) for _ in range(2)],
          *[pltpu.VMEM((CF_DIM, PITCH), jnp.float32) for _ in range(2)],
          *[pltpu.SemaphoreType.DMA for _ in range(4)],
      ],
  )
  def k(idx_hbm, table_hbm, out_hbm, idx_v, rows0, rows1, t0, t1,
        g0, g1, o0, o1):
    rows = (rows0, rows1)
    tiles = (t0, t1)
    gsem = (g0, g1)
    osem = (o0, o1)
    wid = lax.axis_index("s") * NC + lax.axis_index("c")
    base_blk = wid * BLK_PER_W

    pltpu.sync_copy(idx_hbm.at[pl.ds(base_blk, BLK_PER_W)], idx_v)

    iot = lax.iota(jnp.int32, 16)
    rvecs = [iot + 16 * dg for dg in range(4)]

    def transpose_block(src, dst):
      for b0 in range(0, BLK, 4):
        vs = [
            src[b0 + i, pl.ds(16 * dg, 16)]
            for i in range(4)
            for dg in range(4)
        ]
        for i in range(4):
          bvec = jnp.full((16,), b0 + i, jnp.int32)
          for dg in range(4):
            plsc.store_scatter(dst, [rvecs[dg], bvec], vs[i * 4 + dg])

    def out_copies(src, s, bh, sem):
      for dh in range(8):
        pltpu.async_copy(
            src.at[pl.ds(8 * dh, 8), pl.ds(0, BLK)],
            out_hbm.at[s, dh, bh], sem)

    def drain_out(src, sem):
      for dh in range(8):
        pltpu.make_async_copy(
            src.at[pl.ds(8 * dh, 8), pl.ds(0, BLK)],
            out_hbm.at[0, 0, 0], sem).wait()

    pltpu.async_copy(table_hbm.at[idx_v.at[0]], rows[0], gsem[0])

    @pl.loop(0, BLK_PER_W // 2)
    def _(g):
      for p in range(2):
        j = 2 * g + p
        @pl.when(j + 1 < BLK_PER_W)
        def _():
          pltpu.async_copy(
              table_hbm.at[idx_v.at[j + 1]], rows[1 - p], gsem[1 - p])

        pltpu.make_async_copy(
            table_hbm.at[pl.ds(0, BLK)], rows[p], gsem[p]).wait()

        @pl.when(j >= 2)
        def _():
          drain_out(tiles[p], osem[p])

        transpose_block(rows[p], tiles[p])

        blk = base_blk + j
        s = blk // BH
        bh = lax.rem(blk, BH)
        out_copies(tiles[p], s, bh, osem[p])

    for p in range(2):
      drain_out(tiles[p], osem[p])

  return k(idx2, table)


@jax.jit
def kernel(item_indices, item_embeddings):
  idx2 = item_indices.astype(jnp.int32).T.reshape(N_BLOCKS, BLK)
  out5 = _sc_gather(idx2, item_embeddings)
  return out5.transpose((2, 4, 0, 1, 3)).reshape(NB, NSLOT, CF_DIM)
